# Initial kernel scaffold; baseline (speedup 1.0000x reference)
import jax
import jax.numpy as jnp
from jax.experimental import pallas as pl


def kernel(pred_logits, pred_boxes, gt_boxes, gt_labels, grid_size, image_size, offset):
    raise NotImplementedError("write your pallas kernel here")


# speedup vs baseline: 8.3841x; 8.3841x over previous
import jax
import jax.numpy as jnp
from jax.experimental import pallas as pl
from jax.experimental.pallas import tpu as pltpu

_COST_CLASS = 1.0
_COST_BBOX = 1.0
_COST_GIOU = 1.0
_OTA_K = 10
_CENTER_RADIUS = 2.5
_ALPHA = 0.25

_POS_INF = float("inf")
_NEG_INF = float("-inf")
_BIG_I32 = 2**31 - 1


def _cost_and_iou_block(logits, pbox, gtT, labels_row, gs, isz, off):
    R, C = logits.shape
    M = labels_row.shape[1]

    p = jax.nn.sigmoid(logits)
    omp = 1.0 - p
    neg = (1.0 - _ALPHA) * (p * p) * (-jnp.log(1.0 - p + 1e-8))
    pos = _ALPHA * (omp * omp) * (-jnp.log(p + 1e-8))
    diff = pos - neg
    cost_class = jnp.zeros((R, M), jnp.float32)
    for c in range(C):
        cost_class = jnp.where(labels_row == c, diff[:, c:c + 1], cost_class)

    pcol = [pbox[:, k:k + 1] for k in range(7)]
    grow = [gtT[k:k + 1, :] for k in range(7)]

    nb = [pcol[0] / gs[0], pcol[1] / gs[1], pcol[2] / gs[2],
          pcol[3] / gs[0], pcol[4] / gs[1], pcol[5] / gs[2], pcol[6]]
    tgt = [(grow[0] - off[0]) / isz[0], (grow[1] - off[1]) / isz[1],
           (grow[2] - off[2]) / isz[2],
           grow[3] / isz[0], grow[4] / isz[1], grow[5] / isz[2], grow[6]]
    tgt_bev = [tgt[0] * gs[0], tgt[1] * gs[1], tgt[2] * gs[2],
               tgt[3] * gs[0], tgt[4] * gs[1], tgt[5] * gs[2], tgt[6]]
    wx = nb[0] * isz[0]
    wy = nb[1] * isz[1]

    ca = jnp.cos(grow[6])
    sa = jnp.sin(grow[6])
    hx = grow[3] / 2.0
    hy = grow[4] / 2.0
    x0 = (ca * (-hx) + sa * (-hy)) + grow[0]
    y0 = (-sa * (-hx) + ca * (-hy)) + grow[1]
    x1 = (ca * (-hx) + sa * hy) + grow[0]
    y1 = (-sa * (-hx) + ca * hy) + grow[1]
    x3 = (ca * hx + sa * (-hy)) + grow[0]
    y3 = (-sa * hx + ca * (-hy)) + grow[1]

    ab0 = x1 - x0
    ab1 = y1 - y0
    ad0 = x3 - x0
    ad1 = y3 - y0
    ap0 = wx - x0
    ap1 = wy - y0
    abab = ab0 * ab0 + ab1 * ab1
    abap = ab0 * ap0 + ab1 * ap1
    adad = ad0 * ad0 + ad1 * ad1
    adap = ad0 * ap0 + ad1 * ap1
    one = jnp.ones((), jnp.int32)
    zero = jnp.zeros((), jnp.int32)
    cnt = (jnp.where(abab >= abap, one, zero) + jnp.where(abap >= 0, one, zero)
           + jnp.where(adad >= adap, one, zero)
           + jnp.where(adap >= 0, one, zero))
    in_boxes = cnt == 4

    dx = jnp.abs(wx - grow[0])
    dy = jnp.abs(wy - grow[1])
    in_centers = (dx < _CENTER_RADIUS) & (dy < _CENTER_RADIUS)
    fg = jnp.max(jnp.where(in_boxes | in_centers, 1.0, 0.0), axis=1,
                 keepdims=True)
    in_both = in_boxes & in_centers

    cost_bbox = jnp.abs(nb[0] - tgt[0])
    for k in range(1, 7):
        cost_bbox = cost_bbox + jnp.abs(nb[k] - tgt[k])

    inter = None
    for k in range(3):
        amin = pcol[k] - pcol[3 + k] / 2.0
        amax = pcol[k] + pcol[3 + k] / 2.0
        bmin = tgt_bev[k] - tgt_bev[3 + k] / 2.0
        bmax = tgt_bev[k] + tgt_bev[3 + k] / 2.0
        d = jnp.maximum(jnp.minimum(amax, bmax) - jnp.maximum(amin, bmin),
                        0.0)
        inter = d if inter is None else inter * d
    va = (jnp.maximum(pcol[3], 1e-6) * jnp.maximum(pcol[4], 1e-6)
          * jnp.maximum(pcol[5], 1e-6))
    vb = (jnp.maximum(tgt_bev[3], 1e-6) * jnp.maximum(tgt_bev[4], 1e-6)
          * jnp.maximum(tgt_bev[5], 1e-6))
    iou = inter / (va + vb - inter + 1e-8)

    cost = (_COST_BBOX * cost_bbox + _COST_CLASS * cost_class
            - _COST_GIOU * iou
            + 100.0 * jnp.where(in_both, 0.0, 1.0))
    cost = cost + 10000.0 * (1.0 - fg)
    return cost, iou


def _read_scalars(gs_ref, isz_ref, off_ref):
    return ((gs_ref[0], gs_ref[1], gs_ref[2]),
            (isz_ref[0], isz_ref[1], isz_ref[2]),
            (off_ref[0], off_ref[1], off_ref[2]))


def _pass1(logits_ref, pbox_ref, gtT_ref, labels_ref, gs_ref, isz_ref,
           off_ref, tv_ref, ti_ref, sv_ref, si_ref, iv_ref, ii_ref):
    pid = pl.program_id(0)
    nblk = pl.num_programs(0)
    R = logits_ref.shape[0]
    M = labels_ref.shape[1]

    @pl.when(pid == 0)
    def _init():
        sv_ref[...] = jnp.full((16, M), _POS_INF, jnp.float32)
        si_ref[...] = jnp.full((16, M), _BIG_I32, jnp.int32)
        iv_ref[...] = jnp.full((16, M), _NEG_INF, jnp.float32)
        ii_ref[...] = jnp.full((16, M), _BIG_I32, jnp.int32)

    gs, isz, off = _read_scalars(gs_ref, isz_ref, off_ref)
    cost, iou = _cost_and_iou_block(logits_ref[...], pbox_ref[...],
                                    gtT_ref[...], labels_ref[0:1, :],
                                    gs, isz, off)
    ridx = jax.lax.broadcasted_iota(jnp.int32, (R, M), 0) + pid * R

    wv = jnp.concatenate([sv_ref[...], cost], axis=0)
    wi = jnp.concatenate([si_ref[...], ridx], axis=0)
    cvals, cidxs = [], []
    for _ in range(_OTA_K):
        m = jnp.min(wv, axis=0, keepdims=True)
        eq = wv == m
        mi = jnp.min(jnp.where(eq, wi, _BIG_I32), axis=0, keepdims=True)
        cvals.append(m)
        cidxs.append(mi)
        wv = jnp.where(eq & (wi == mi), _POS_INF, wv)
    sv_ref[...] = jnp.concatenate(
        cvals + [jnp.full((16 - _OTA_K, M), _POS_INF, jnp.float32)], axis=0)
    si_ref[...] = jnp.concatenate(
        cidxs + [jnp.full((16 - _OTA_K, M), _BIG_I32, jnp.int32)], axis=0)

    uv = jnp.concatenate([iv_ref[...], iou], axis=0)
    ui = jnp.concatenate([ii_ref[...], ridx], axis=0)
    ivals = []
    for _ in range(_OTA_K):
        m = jnp.max(uv, axis=0, keepdims=True)
        eq = uv == m
        mi = jnp.min(jnp.where(eq, ui, _BIG_I32), axis=0, keepdims=True)
        ivals.append(m)
        uv = jnp.where(eq & (ui == mi), _NEG_INF, uv)
    iv_ref[...] = jnp.concatenate(
        ivals + [jnp.full((16 - _OTA_K, M), _NEG_INF, jnp.float32)], axis=0)
    ii_ref[...] = jnp.full((16, M), _BIG_I32, jnp.int32)

    @pl.when(pid == nblk - 1)
    def _finish():
        s = ivals[0]
        for t in range(1, _OTA_K):
            s = s + ivals[t]
        k = jnp.clip(s.astype(jnp.int32), 1, None)
        km1 = k - 1
        tv = jnp.zeros((1, M), jnp.float32)
        ti = jnp.zeros((1, M), jnp.int32)
        for t in range(_OTA_K):
            sel = km1 == t
            tv = jnp.where(sel, cvals[t], tv)
            ti = jnp.where(sel, cidxs[t], ti)
        tv_ref[...] = jnp.broadcast_to(tv, (8, M))
        ti_ref[...] = jnp.broadcast_to(ti, (8, M))


def _pass2(logits_ref, pbox_ref, gtT_ref, labels_ref, gs_ref, isz_ref,
           off_ref, tv_ref, ti_ref, match_ref, cnt_ref, cs_ref):
    pid = pl.program_id(0)
    nblk = pl.num_programs(0)
    R = logits_ref.shape[0]
    M = labels_ref.shape[1]

    @pl.when(pid == 0)
    def _init():
        cs_ref[...] = jnp.zeros((8, M), jnp.float32)

    gs, isz, off = _read_scalars(gs_ref, isz_ref, off_ref)
    cost, _ = _cost_and_iou_block(logits_ref[...], pbox_ref[...],
                                  gtT_ref[...], labels_ref[0:1, :],
                                  gs, isz, off)
    ridx = jax.lax.broadcasted_iota(jnp.int32, (R, M), 0) + pid * R
    tv = tv_ref[0:1, :]
    ti = ti_ref[0:1, :]

    m0 = jnp.where((cost < tv) | ((cost == tv) & (ridx <= ti)), 1.0, 0.0)
    am = jnp.sum(m0, axis=1, keepdims=True)

    rowmin = jnp.min(cost, axis=1, keepdims=True)
    lane = jax.lax.broadcasted_iota(jnp.int32, (R, M), 1)
    bidx = jnp.min(jnp.where(cost == rowmin, lane, _BIG_I32), axis=1,
                   keepdims=True)
    onehot = jnp.where(lane == bidx, 1.0, 0.0)

    mat = jnp.where(am > 1.0, onehot, m0)
    match_ref[...] = mat
    cs_ref[0:1, :] = cs_ref[0:1, :] + jnp.sum(mat, axis=0, keepdims=True)

    @pl.when(pid == nblk - 1)
    def _finish():
        cnt = jnp.sum(
            jnp.where(cs_ref[0:1, :] > 0.0, jnp.ones((), jnp.int32),
                      jnp.zeros((), jnp.int32)), axis=1, keepdims=True)
        cnt_ref[...] = jnp.broadcast_to(cnt, (8, M))


def _pick_block(nq):
    for r in (2000, 2500, 1000, 800, 500, 400, 250, 200, 125, 100, 50, 40,
              25, 20, 10, 8, 5, 4, 2, 1):
        if nq % r == 0:
            return r
    return nq


def kernel(pred_logits, pred_boxes, gt_boxes, gt_labels, grid_size,
           image_size, offset):
    nq, _ = pred_logits.shape
    m = gt_boxes.shape[0]
    r = _pick_block(nq)
    nblk = nq // r

    gtT = jnp.concatenate(
        [gt_boxes.T.astype(jnp.float32), jnp.zeros((1, m), jnp.float32)],
        axis=0)
    labels2d = jnp.broadcast_to(
        gt_labels.astype(jnp.int32).reshape(1, m), (8, m))
    gs = grid_size.astype(jnp.float32)
    isz = image_size.astype(jnp.float32)
    off = offset.astype(jnp.float32)

    common_in = [
        pl.BlockSpec((r, pred_logits.shape[1]), lambda i: (i, 0)),
        pl.BlockSpec((r, 7), lambda i: (i, 0)),
        pl.BlockSpec((8, m), lambda i: (0, 0)),
        pl.BlockSpec((8, m), lambda i: (0, 0)),
        pl.BlockSpec(memory_space=pltpu.SMEM),
        pl.BlockSpec(memory_space=pltpu.SMEM),
        pl.BlockSpec(memory_space=pltpu.SMEM),
    ]

    tv, ti = pl.pallas_call(
        _pass1,
        grid=(nblk,),
        in_specs=common_in,
        out_specs=[pl.BlockSpec((8, m), lambda i: (0, 0)),
                   pl.BlockSpec((8, m), lambda i: (0, 0))],
        out_shape=[jax.ShapeDtypeStruct((8, m), jnp.float32),
                   jax.ShapeDtypeStruct((8, m), jnp.int32)],
        scratch_shapes=[pltpu.VMEM((16, m), jnp.float32),
                        pltpu.VMEM((16, m), jnp.int32),
                        pltpu.VMEM((16, m), jnp.float32),
                        pltpu.VMEM((16, m), jnp.int32)],
    )(pred_logits, pred_boxes, gtT, labels2d, gs, isz, off)

    matching, cnt = pl.pallas_call(
        _pass2,
        grid=(nblk,),
        in_specs=common_in + [pl.BlockSpec((8, m), lambda i: (0, 0)),
                              pl.BlockSpec((8, m), lambda i: (0, 0))],
        out_specs=[pl.BlockSpec((r, m), lambda i: (i, 0)),
                   pl.BlockSpec((8, m), lambda i: (0, 0))],
        out_shape=[jax.ShapeDtypeStruct((nq, m), jnp.float32),
                   jax.ShapeDtypeStruct((8, m), jnp.int32)],
        scratch_shapes=[pltpu.VMEM((8, m), jnp.float32)],
    )(pred_logits, pred_boxes, gtT, labels2d, gs, isz, off, tv, ti)

    return matching, cnt[0, 0]


# speedup vs baseline: 14.3226x; 1.7083x over previous
import jax
import jax.numpy as jnp
from jax.experimental import pallas as pl
from jax.experimental.pallas import tpu as pltpu

_COST_CLASS = 1.0
_COST_BBOX = 1.0
_COST_GIOU = 1.0
_OTA_K = 10
_CENTER_RADIUS = 2.5
_ALPHA = 0.25

_POS_INF = float("inf")
_NEG_INF = float("-inf")
_BIG_I32 = 2**31 - 1


def _cost_and_iou_block(logits, pbox, gtT, labels_row, gs, isz, off):
    R, C = logits.shape
    M = labels_row.shape[1]

    p = jax.nn.sigmoid(logits)
    omp = 1.0 - p
    neg = (1.0 - _ALPHA) * (p * p) * (-jnp.log(1.0 - p + 1e-8))
    pos = _ALPHA * (omp * omp) * (-jnp.log(p + 1e-8))
    diff = pos - neg
    cpad = 16
    diff16 = jnp.concatenate(
        [diff, jnp.zeros((R, cpad - C), jnp.float32)], axis=1)
    onehot_lab = jnp.where(
        jax.lax.broadcasted_iota(jnp.int32, (cpad, M), 0) == labels_row,
        1.0, 0.0)
    cost_class = jax.lax.dot_general(
        diff16, onehot_lab, (((1,), (0,)), ((), ())),
        precision=jax.lax.Precision.HIGHEST,
        preferred_element_type=jnp.float32)

    pcol = [pbox[:, k:k + 1] for k in range(7)]
    grow = [gtT[k:k + 1, :] for k in range(7)]

    nb = [pcol[0] / gs[0], pcol[1] / gs[1], pcol[2] / gs[2],
          pcol[3] / gs[0], pcol[4] / gs[1], pcol[5] / gs[2], pcol[6]]
    tgt = [(grow[0] - off[0]) / isz[0], (grow[1] - off[1]) / isz[1],
           (grow[2] - off[2]) / isz[2],
           grow[3] / isz[0], grow[4] / isz[1], grow[5] / isz[2], grow[6]]
    tgt_bev = [tgt[0] * gs[0], tgt[1] * gs[1], tgt[2] * gs[2],
               tgt[3] * gs[0], tgt[4] * gs[1], tgt[5] * gs[2], tgt[6]]
    wx = nb[0] * isz[0]
    wy = nb[1] * isz[1]

    ca = jnp.cos(grow[6])
    sa = jnp.sin(grow[6])
    hx = grow[3] / 2.0
    hy = grow[4] / 2.0
    x0 = (ca * (-hx) + sa * (-hy)) + grow[0]
    y0 = (-sa * (-hx) + ca * (-hy)) + grow[1]
    x1 = (ca * (-hx) + sa * hy) + grow[0]
    y1 = (-sa * (-hx) + ca * hy) + grow[1]
    x3 = (ca * hx + sa * (-hy)) + grow[0]
    y3 = (-sa * hx + ca * (-hy)) + grow[1]

    ab0 = x1 - x0
    ab1 = y1 - y0
    ad0 = x3 - x0
    ad1 = y3 - y0
    ap0 = wx - x0
    ap1 = wy - y0
    abab = ab0 * ab0 + ab1 * ab1
    abap = ab0 * ap0 + ab1 * ap1
    adad = ad0 * ad0 + ad1 * ad1
    adap = ad0 * ap0 + ad1 * ap1
    in_boxes = ((abab >= abap) & (abap >= 0)
                & ((adad >= adap) & (adap >= 0)))

    dx = jnp.abs(wx - grow[0])
    dy = jnp.abs(wy - grow[1])
    in_centers = (dx < _CENTER_RADIUS) & (dy < _CENTER_RADIUS)
    fg = jnp.max(jnp.where(in_boxes | in_centers, 1.0, 0.0), axis=1,
                 keepdims=True)
    in_both = in_boxes & in_centers

    cost_bbox = jnp.abs(nb[0] - tgt[0])
    for k in range(1, 7):
        cost_bbox = cost_bbox + jnp.abs(nb[k] - tgt[k])

    inter = None
    for k in range(3):
        amin = pcol[k] - pcol[3 + k] / 2.0
        amax = pcol[k] + pcol[3 + k] / 2.0
        bmin = tgt_bev[k] - tgt_bev[3 + k] / 2.0
        bmax = tgt_bev[k] + tgt_bev[3 + k] / 2.0
        d = jnp.maximum(jnp.minimum(amax, bmax) - jnp.maximum(amin, bmin),
                        0.0)
        inter = d if inter is None else inter * d
    va = (jnp.maximum(pcol[3], 1e-6) * jnp.maximum(pcol[4], 1e-6)
          * jnp.maximum(pcol[5], 1e-6))
    vb = (jnp.maximum(tgt_bev[3], 1e-6) * jnp.maximum(tgt_bev[4], 1e-6)
          * jnp.maximum(tgt_bev[5], 1e-6))
    iou = inter / (va + vb - inter + 1e-8)

    cost = (_COST_BBOX * cost_bbox + _COST_CLASS * cost_class
            - _COST_GIOU * iou
            + 100.0 * jnp.where(in_both, 0.0, 1.0))
    cost = cost + 10000.0 * (1.0 - fg)
    return cost, iou


def _read_scalars(gs_ref, isz_ref, off_ref):
    return ((gs_ref[0], gs_ref[1], gs_ref[2]),
            (isz_ref[0], isz_ref[1], isz_ref[2]),
            (off_ref[0], off_ref[1], off_ref[2]))


def _extract_topk(blk, sv, si, base, largest):
    R, M = blk.shape
    iota = jax.lax.broadcasted_iota(jnp.int32, (R, M), 0) + base
    if largest:
        red, sentinel = jnp.max, _NEG_INF
        better = lambda a, b: jnp.maximum(a, b)
    else:
        red, sentinel = jnp.min, _POS_INF
        better = lambda a, b: jnp.minimum(a, b)
    vals, idxs = [], []
    for _ in range(_OTA_K):
        m = better(red(blk, axis=0, keepdims=True),
                   red(sv, axis=0, keepdims=True))
        eq_b = blk == m
        eq_s = sv == m
        mi = jnp.minimum(
            jnp.min(jnp.where(eq_b, iota, _BIG_I32), axis=0, keepdims=True),
            jnp.min(jnp.where(eq_s, si, _BIG_I32), axis=0, keepdims=True))
        vals.append(m)
        idxs.append(mi)
        blk = jnp.where(eq_b & (iota == mi), sentinel, blk)
        sv = jnp.where(eq_s & (si == mi), sentinel, sv)
    return vals, idxs


def _pass1(logits_ref, pbox_ref, gtT_ref, labels_ref, gs_ref, isz_ref,
           off_ref, cost_ref, tv_ref, ti_ref, sv_ref, si_ref, iv_ref,
           ii_ref):
    pid = pl.program_id(0)
    nblk = pl.num_programs(0)
    R = logits_ref.shape[0]
    M = labels_ref.shape[1]

    @pl.when(pid == 0)
    def _init():
        sv_ref[...] = jnp.full((16, M), _POS_INF, jnp.float32)
        si_ref[...] = jnp.full((16, M), _BIG_I32, jnp.int32)
        iv_ref[...] = jnp.full((16, M), _NEG_INF, jnp.float32)
        ii_ref[...] = jnp.full((16, M), _BIG_I32, jnp.int32)

    gs, isz, off = _read_scalars(gs_ref, isz_ref, off_ref)
    cost, iou = _cost_and_iou_block(logits_ref[...], pbox_ref[...],
                                    gtT_ref[...], labels_ref[0:1, :],
                                    gs, isz, off)
    cost_ref[...] = cost

    cvals, cidxs = _extract_topk(cost, sv_ref[...], si_ref[...], pid * R,
                                 largest=False)
    pad_v = jnp.full((16 - _OTA_K, M), _POS_INF, jnp.float32)
    pad_i = jnp.full((16 - _OTA_K, M), _BIG_I32, jnp.int32)
    sv_ref[...] = jnp.concatenate(cvals + [pad_v], axis=0)
    si_ref[...] = jnp.concatenate(cidxs + [pad_i], axis=0)

    ivals, iidxs = _extract_topk(iou, iv_ref[...], ii_ref[...], pid * R,
                                 largest=True)
    iv_ref[...] = jnp.concatenate(
        ivals + [jnp.full((16 - _OTA_K, M), _NEG_INF, jnp.float32)], axis=0)
    ii_ref[...] = jnp.concatenate(iidxs + [pad_i], axis=0)

    @pl.when(pid == nblk - 1)
    def _finish():
        s = ivals[0]
        for t in range(1, _OTA_K):
            s = s + ivals[t]
        k = jnp.clip(s.astype(jnp.int32), 1, None)
        km1 = k - 1
        tv = jnp.zeros((1, M), jnp.float32)
        ti = jnp.zeros((1, M), jnp.int32)
        for t in range(_OTA_K):
            sel = km1 == t
            tv = jnp.where(sel, cvals[t], tv)
            ti = jnp.where(sel, cidxs[t], ti)
        tv_ref[...] = jnp.broadcast_to(tv, (8, M))
        ti_ref[...] = jnp.broadcast_to(ti, (8, M))


def _pass2(cost_ref, tv_ref, ti_ref, match_ref, cnt_ref, cs_ref):
    pid = pl.program_id(0)
    nblk = pl.num_programs(0)
    R, M = cost_ref.shape

    @pl.when(pid == 0)
    def _init():
        cs_ref[...] = jnp.zeros((8, M), jnp.float32)

    cost = cost_ref[...]
    ridx = jax.lax.broadcasted_iota(jnp.int32, (R, M), 0) + pid * R
    tv = tv_ref[0:1, :]
    ti = ti_ref[0:1, :]

    m0 = jnp.where((cost < tv) | ((cost == tv) & (ridx <= ti)), 1.0, 0.0)
    am = jnp.sum(m0, axis=1, keepdims=True)

    rowmin = jnp.min(cost, axis=1, keepdims=True)
    lane = jax.lax.broadcasted_iota(jnp.int32, (R, M), 1)
    bidx = jnp.min(jnp.where(cost == rowmin, lane, _BIG_I32), axis=1,
                   keepdims=True)
    onehot = jnp.where(lane == bidx, 1.0, 0.0)

    mat = jnp.where(am > 1.0, onehot, m0)
    match_ref[...] = mat
    cs_ref[0:1, :] = cs_ref[0:1, :] + jnp.sum(mat, axis=0, keepdims=True)

    @pl.when(pid == nblk - 1)
    def _finish():
        cnt = jnp.sum(
            jnp.where(cs_ref[0:1, :] > 0.0, jnp.ones((), jnp.int32),
                      jnp.zeros((), jnp.int32)), axis=1, keepdims=True)
        cnt_ref[...] = jnp.broadcast_to(cnt, (8, M))


def _pick_block(nq):
    for r in (2000, 2500, 1000, 800, 500, 400, 250, 200, 125, 100, 50, 40,
              25, 20, 10, 8, 5, 4, 2, 1):
        if nq % r == 0:
            return r
    return nq


def kernel(pred_logits, pred_boxes, gt_boxes, gt_labels, grid_size,
           image_size, offset):
    nq, _ = pred_logits.shape
    m = gt_boxes.shape[0]
    r = _pick_block(nq)
    nblk = nq // r

    gtT = jnp.concatenate(
        [gt_boxes.T.astype(jnp.float32), jnp.zeros((1, m), jnp.float32)],
        axis=0)
    labels2d = jnp.broadcast_to(
        gt_labels.astype(jnp.int32).reshape(1, m), (8, m))
    gs = grid_size.astype(jnp.float32)
    isz = image_size.astype(jnp.float32)
    off = offset.astype(jnp.float32)

    cost, tv, ti = pl.pallas_call(
        _pass1,
        grid=(nblk,),
        in_specs=[
            pl.BlockSpec((r, pred_logits.shape[1]), lambda i: (i, 0)),
            pl.BlockSpec((r, 7), lambda i: (i, 0)),
            pl.BlockSpec((8, m), lambda i: (0, 0)),
            pl.BlockSpec((8, m), lambda i: (0, 0)),
            pl.BlockSpec(memory_space=pltpu.SMEM),
            pl.BlockSpec(memory_space=pltpu.SMEM),
            pl.BlockSpec(memory_space=pltpu.SMEM),
        ],
        out_specs=[pl.BlockSpec((r, m), lambda i: (i, 0)),
                   pl.BlockSpec((8, m), lambda i: (0, 0)),
                   pl.BlockSpec((8, m), lambda i: (0, 0))],
        out_shape=[jax.ShapeDtypeStruct((nq, m), jnp.float32),
                   jax.ShapeDtypeStruct((8, m), jnp.float32),
                   jax.ShapeDtypeStruct((8, m), jnp.int32)],
        scratch_shapes=[pltpu.VMEM((16, m), jnp.float32),
                        pltpu.VMEM((16, m), jnp.int32),
                        pltpu.VMEM((16, m), jnp.float32),
                        pltpu.VMEM((16, m), jnp.int32)],
    )(pred_logits, pred_boxes, gtT, labels2d, gs, isz, off)

    matching, cnt = pl.pallas_call(
        _pass2,
        grid=(nblk,),
        in_specs=[pl.BlockSpec((r, m), lambda i: (i, 0)),
                  pl.BlockSpec((8, m), lambda i: (0, 0)),
                  pl.BlockSpec((8, m), lambda i: (0, 0))],
        out_specs=[pl.BlockSpec((r, m), lambda i: (i, 0)),
                   pl.BlockSpec((8, m), lambda i: (0, 0))],
        out_shape=[jax.ShapeDtypeStruct((nq, m), jnp.float32),
                   jax.ShapeDtypeStruct((8, m), jnp.int32)],
        scratch_shapes=[pltpu.VMEM((8, m), jnp.float32)],
    )(cost, tv, ti)

    return matching, cnt[0, 0]


# speedup vs baseline: 16.0832x; 1.1229x over previous
import jax
import jax.numpy as jnp
from jax.experimental import pallas as pl
from jax.experimental.pallas import tpu as pltpu

_COST_CLASS = 1.0
_COST_BBOX = 1.0
_COST_GIOU = 1.0
_OTA_K = 10
_CENTER_RADIUS = 2.5
_ALPHA = 0.25

_POS_INF = float("inf")
_NEG_INF = float("-inf")
_BIG_I32 = 2**31 - 1


def _cost_and_iou_block(logits, pbox, gtT, labels_row, gs, isz, off):
    R, C = logits.shape
    M = labels_row.shape[1]

    p = jax.nn.sigmoid(logits)
    omp = 1.0 - p
    neg = (1.0 - _ALPHA) * (p * p) * (-jnp.log(1.0 - p + 1e-8))
    pos = _ALPHA * (omp * omp) * (-jnp.log(p + 1e-8))
    diff = pos - neg
    cpad = 16
    diff16 = jnp.concatenate(
        [diff, jnp.zeros((R, cpad - C), jnp.float32)], axis=1)
    onehot_lab = jnp.where(
        jax.lax.broadcasted_iota(jnp.int32, (cpad, M), 0) == labels_row,
        1.0, 0.0)
    cost_class = jax.lax.dot_general(
        diff16, onehot_lab, (((1,), (0,)), ((), ())),
        precision=jax.lax.Precision.HIGHEST,
        preferred_element_type=jnp.float32)

    pcol = [pbox[:, k:k + 1] for k in range(7)]
    grow = [gtT[k:k + 1, :] for k in range(7)]

    nb = [pcol[0] / gs[0], pcol[1] / gs[1], pcol[2] / gs[2],
          pcol[3] / gs[0], pcol[4] / gs[1], pcol[5] / gs[2], pcol[6]]
    tgt = [(grow[0] - off[0]) / isz[0], (grow[1] - off[1]) / isz[1],
           (grow[2] - off[2]) / isz[2],
           grow[3] / isz[0], grow[4] / isz[1], grow[5] / isz[2], grow[6]]
    tgt_bev = [tgt[0] * gs[0], tgt[1] * gs[1], tgt[2] * gs[2],
               tgt[3] * gs[0], tgt[4] * gs[1], tgt[5] * gs[2], tgt[6]]
    wx = nb[0] * isz[0]
    wy = nb[1] * isz[1]

    ca = jnp.cos(grow[6])
    sa = jnp.sin(grow[6])
    hx = grow[3] / 2.0
    hy = grow[4] / 2.0
    x0 = (ca * (-hx) + sa * (-hy)) + grow[0]
    y0 = (-sa * (-hx) + ca * (-hy)) + grow[1]
    x1 = (ca * (-hx) + sa * hy) + grow[0]
    y1 = (-sa * (-hx) + ca * hy) + grow[1]
    x3 = (ca * hx + sa * (-hy)) + grow[0]
    y3 = (-sa * hx + ca * (-hy)) + grow[1]

    ab0 = x1 - x0
    ab1 = y1 - y0
    ad0 = x3 - x0
    ad1 = y3 - y0
    ap0 = wx - x0
    ap1 = wy - y0
    abab = ab0 * ab0 + ab1 * ab1
    abap = ab0 * ap0 + ab1 * ap1
    adad = ad0 * ad0 + ad1 * ad1
    adap = ad0 * ap0 + ad1 * ap1
    in_boxes = ((abab >= abap) & (abap >= 0)
                & ((adad >= adap) & (adap >= 0)))

    dx = jnp.abs(wx - grow[0])
    dy = jnp.abs(wy - grow[1])
    in_centers = (dx < _CENTER_RADIUS) & (dy < _CENTER_RADIUS)
    fg = jnp.max(jnp.where(in_boxes | in_centers, 1.0, 0.0), axis=1,
                 keepdims=True)
    in_both = in_boxes & in_centers

    cost_bbox = jnp.abs(nb[0] - tgt[0])
    for k in range(1, 7):
        cost_bbox = cost_bbox + jnp.abs(nb[k] - tgt[k])

    inter = None
    for k in range(3):
        amin = pcol[k] - pcol[3 + k] / 2.0
        amax = pcol[k] + pcol[3 + k] / 2.0
        bmin = tgt_bev[k] - tgt_bev[3 + k] / 2.0
        bmax = tgt_bev[k] + tgt_bev[3 + k] / 2.0
        d = jnp.maximum(jnp.minimum(amax, bmax) - jnp.maximum(amin, bmin),
                        0.0)
        inter = d if inter is None else inter * d
    va = (jnp.maximum(pcol[3], 1e-6) * jnp.maximum(pcol[4], 1e-6)
          * jnp.maximum(pcol[5], 1e-6))
    vb = (jnp.maximum(tgt_bev[3], 1e-6) * jnp.maximum(tgt_bev[4], 1e-6)
          * jnp.maximum(tgt_bev[5], 1e-6))
    iou = inter / (va + vb - inter + 1e-8)

    cost = (_COST_BBOX * cost_bbox + _COST_CLASS * cost_class
            - _COST_GIOU * iou
            + 100.0 * jnp.where(in_both, 0.0, 1.0))
    cost = cost + 10000.0 * (1.0 - fg)
    return cost, iou


def _extract_topk(blk, sv, si, base, largest):
    R, M = blk.shape
    iota = jax.lax.broadcasted_iota(jnp.int32, (R, M), 0) + base
    if largest:
        red, sentinel = jnp.max, _NEG_INF
        better = jnp.maximum
    else:
        red, sentinel = jnp.min, _POS_INF
        better = jnp.minimum
    vals, idxs = [], []
    for _ in range(_OTA_K):
        m = better(red(blk, axis=0, keepdims=True),
                   red(sv, axis=0, keepdims=True))
        eq_b = blk == m
        eq_s = sv == m
        mi = jnp.minimum(
            jnp.min(jnp.where(eq_b, iota, _BIG_I32), axis=0, keepdims=True),
            jnp.min(jnp.where(eq_s, si, _BIG_I32), axis=0, keepdims=True))
        vals.append(m)
        idxs.append(mi)
        blk = jnp.where(eq_b & (iota == mi), sentinel, blk)
        sv = jnp.where(eq_s & (si == mi), sentinel, sv)
    return vals, idxs


def _fused(logits_ref, pbox_ref, gtT_ref, labels_ref, gs_ref, isz_ref,
           off_ref, match_ref, cnt_ref, costs_ref, kq_ref, iv_ref, ii_ref,
           tvs_ref, tis_ref, cs_ref):
    ph = pl.program_id(0)
    pid = pl.program_id(1)
    nblk = pl.num_programs(1)
    R = logits_ref.shape[0]
    NQ, M = costs_ref.shape

    @pl.when(ph == 0)
    def _phase0():
        @pl.when(pid == 0)
        def _init():
            iv_ref[...] = jnp.full((16, M), _NEG_INF, jnp.float32)
            ii_ref[...] = jnp.full((16, M), _BIG_I32, jnp.int32)

        gs = (gs_ref[0], gs_ref[1], gs_ref[2])
        isz = (isz_ref[0], isz_ref[1], isz_ref[2])
        off = (off_ref[0], off_ref[1], off_ref[2])
        cost, iou = _cost_and_iou_block(logits_ref[...], pbox_ref[...],
                                        gtT_ref[...], labels_ref[0:1, :],
                                        gs, isz, off)
        costs_ref[pl.ds(pid * R, R), :] = cost

        ivals, iidxs = _extract_topk(iou, iv_ref[...], ii_ref[...],
                                     pid * R, largest=True)
        iv_ref[...] = jnp.concatenate(
            ivals + [jnp.full((16 - _OTA_K, M), _NEG_INF, jnp.float32)],
            axis=0)
        ii_ref[...] = jnp.concatenate(
            iidxs + [jnp.full((16 - _OTA_K, M), _BIG_I32, jnp.int32)],
            axis=0)

        @pl.when(pid == nblk - 1)
        def _finish():
            s = ivals[0]
            for t in range(1, _OTA_K):
                s = s + ivals[t]
            k = jnp.clip(s.astype(jnp.int32), 1, None)
            kq_ref[...] = jnp.broadcast_to(k - 1, (8, M))

    @pl.when(ph == 1)
    def _phase1():
        @pl.when(pid == 0)
        def _thresholds():
            cs_ref[...] = jnp.zeros((8, M), jnp.float32)
            km1 = kq_ref[0:1, :]
            kmax = jnp.max(km1) + 1
            iota = jax.lax.broadcasted_iota(jnp.int32, (NQ, M), 0)

            def body(t, carry):
                tvp, tip, tv, ti = carry
                c = costs_ref[...]
                after = (c > tvp) | ((c == tvp) & (iota > tip))
                m = jnp.min(jnp.where(after, c, _POS_INF), axis=0,
                            keepdims=True)
                mi = jnp.min(jnp.where(after & (c == m), iota, _BIG_I32),
                             axis=0, keepdims=True)
                sel = km1 == t
                return (m, mi, jnp.where(sel, m, tv),
                        jnp.where(sel, mi, ti))

            _, _, tv, ti = jax.lax.fori_loop(
                0, kmax, body,
                (jnp.full((1, M), _NEG_INF, jnp.float32),
                 jnp.full((1, M), -1, jnp.int32),
                 jnp.zeros((1, M), jnp.float32),
                 jnp.zeros((1, M), jnp.int32)))
            tvs_ref[...] = jnp.broadcast_to(tv, (8, M))
            tis_ref[...] = jnp.broadcast_to(ti, (8, M))

        cost = costs_ref[pl.ds(pid * R, R), :]
        ridx = jax.lax.broadcasted_iota(jnp.int32, (R, M), 0) + pid * R
        tv = tvs_ref[0:1, :]
        ti = tis_ref[0:1, :]

        m0 = jnp.where((cost < tv) | ((cost == tv) & (ridx <= ti)),
                       1.0, 0.0)
        am = jnp.sum(m0, axis=1, keepdims=True)

        rowmin = jnp.min(cost, axis=1, keepdims=True)
        lane = jax.lax.broadcasted_iota(jnp.int32, (R, M), 1)
        bidx = jnp.min(jnp.where(cost == rowmin, lane, _BIG_I32), axis=1,
                       keepdims=True)
        onehot = jnp.where(lane == bidx, 1.0, 0.0)

        mat = jnp.where(am > 1.0, onehot, m0)
        match_ref[...] = mat
        cs_ref[0:1, :] = cs_ref[0:1, :] + jnp.sum(mat, axis=0,
                                                  keepdims=True)

        @pl.when(pid == nblk - 1)
        def _count():
            cnt = jnp.sum(
                jnp.where(cs_ref[0:1, :] > 0.0, jnp.ones((), jnp.int32),
                          jnp.zeros((), jnp.int32)), axis=1, keepdims=True)
            cnt_ref[...] = jnp.broadcast_to(cnt, (8, M))


def _pick_block(nq):
    for r in (2000, 1000, 800, 500, 400, 250, 200, 125, 100, 50, 40, 25,
              20, 10, 8, 5, 4, 2, 1):
        if nq % r == 0:
            return r
    return nq


def kernel(pred_logits, pred_boxes, gt_boxes, gt_labels, grid_size,
           image_size, offset):
    nq, _ = pred_logits.shape
    m = gt_boxes.shape[0]
    r = _pick_block(nq)
    nblk = nq // r

    gtT = jnp.concatenate(
        [gt_boxes.T.astype(jnp.float32), jnp.zeros((1, m), jnp.float32)],
        axis=0)
    labels2d = jnp.broadcast_to(
        gt_labels.astype(jnp.int32).reshape(1, m), (8, m))
    gs = grid_size.astype(jnp.float32)
    isz = image_size.astype(jnp.float32)
    off = offset.astype(jnp.float32)

    matching, cnt = pl.pallas_call(
        _fused,
        grid=(2, nblk),
        in_specs=[
            pl.BlockSpec((r, pred_logits.shape[1]), lambda p, b: (b, 0)),
            pl.BlockSpec((r, 7), lambda p, b: (b, 0)),
            pl.BlockSpec((8, m), lambda p, b: (0, 0)),
            pl.BlockSpec((8, m), lambda p, b: (0, 0)),
            pl.BlockSpec(memory_space=pltpu.SMEM),
            pl.BlockSpec(memory_space=pltpu.SMEM),
            pl.BlockSpec(memory_space=pltpu.SMEM),
        ],
        out_specs=[pl.BlockSpec((r, m), lambda p, b: (b * p, 0)),
                   pl.BlockSpec((8, m), lambda p, b: (0, 0))],
        out_shape=[jax.ShapeDtypeStruct((nq, m), jnp.float32),
                   jax.ShapeDtypeStruct((8, m), jnp.int32)],
        scratch_shapes=[pltpu.VMEM((nq, m), jnp.float32),
                        pltpu.VMEM((8, m), jnp.int32),
                        pltpu.VMEM((16, m), jnp.float32),
                        pltpu.VMEM((16, m), jnp.int32),
                        pltpu.VMEM((8, m), jnp.float32),
                        pltpu.VMEM((8, m), jnp.int32),
                        pltpu.VMEM((8, m), jnp.float32)],
    )(pred_logits, pred_boxes, gtT, labels2d, gs, isz, off)

    return matching, cnt[0, 0]


# speedup vs baseline: 17.4265x; 1.0835x over previous
import jax
import jax.numpy as jnp
from jax.experimental import pallas as pl
from jax.experimental.pallas import tpu as pltpu

_COST_CLASS = 1.0
_COST_BBOX = 1.0
_COST_GIOU = 1.0
_OTA_K = 10
_CENTER_RADIUS = 2.5
_ALPHA = 0.25

_POS_INF = float("inf")
_NEG_INF = float("-inf")
_BIG_I32 = 2**31 - 1


def _cost_and_iou_block(logits, pbox, gtT, labels_row, gs, isz, off):
    R, C = logits.shape
    M = labels_row.shape[1]

    p = jax.nn.sigmoid(logits)
    omp = 1.0 - p
    neg = (1.0 - _ALPHA) * (p * p) * (-jnp.log(1.0 - p + 1e-8))
    pos = _ALPHA * (omp * omp) * (-jnp.log(p + 1e-8))
    diff = pos - neg
    cpad = 16
    diff16 = jnp.concatenate(
        [diff, jnp.zeros((R, cpad - C), jnp.float32)], axis=1)
    onehot_lab = jnp.where(
        jax.lax.broadcasted_iota(jnp.int32, (cpad, M), 0) == labels_row,
        1.0, 0.0)
    cost_class = jax.lax.dot_general(
        diff16, onehot_lab, (((1,), (0,)), ((), ())),
        precision=jax.lax.Precision.HIGHEST,
        preferred_element_type=jnp.float32)

    pcol = [pbox[:, k:k + 1] for k in range(7)]
    grow = [gtT[k:k + 1, :] for k in range(7)]

    nb = [pcol[0] / gs[0], pcol[1] / gs[1], pcol[2] / gs[2],
          pcol[3] / gs[0], pcol[4] / gs[1], pcol[5] / gs[2], pcol[6]]
    tgt = [(grow[0] - off[0]) / isz[0], (grow[1] - off[1]) / isz[1],
           (grow[2] - off[2]) / isz[2],
           grow[3] / isz[0], grow[4] / isz[1], grow[5] / isz[2], grow[6]]
    tgt_bev = [tgt[0] * gs[0], tgt[1] * gs[1], tgt[2] * gs[2],
               tgt[3] * gs[0], tgt[4] * gs[1], tgt[5] * gs[2], tgt[6]]
    wx = nb[0] * isz[0]
    wy = nb[1] * isz[1]

    ca = jnp.cos(grow[6])
    sa = jnp.sin(grow[6])
    hx = grow[3] / 2.0
    hy = grow[4] / 2.0
    x0 = (ca * (-hx) + sa * (-hy)) + grow[0]
    y0 = (-sa * (-hx) + ca * (-hy)) + grow[1]
    x1 = (ca * (-hx) + sa * hy) + grow[0]
    y1 = (-sa * (-hx) + ca * hy) + grow[1]
    x3 = (ca * hx + sa * (-hy)) + grow[0]
    y3 = (-sa * hx + ca * (-hy)) + grow[1]

    ab0 = x1 - x0
    ab1 = y1 - y0
    ad0 = x3 - x0
    ad1 = y3 - y0
    ap0 = wx - x0
    ap1 = wy - y0
    abab = ab0 * ab0 + ab1 * ab1
    abap = ab0 * ap0 + ab1 * ap1
    adad = ad0 * ad0 + ad1 * ad1
    adap = ad0 * ap0 + ad1 * ap1
    in_boxes = ((abab >= abap) & (abap >= 0)
                & ((adad >= adap) & (adap >= 0)))

    dx = jnp.abs(wx - grow[0])
    dy = jnp.abs(wy - grow[1])
    in_centers = (dx < _CENTER_RADIUS) & (dy < _CENTER_RADIUS)
    fg = jnp.max(jnp.where(in_boxes | in_centers, 1.0, 0.0), axis=1,
                 keepdims=True)
    in_both = in_boxes & in_centers

    cost_bbox = jnp.abs(nb[0] - tgt[0])
    for k in range(1, 7):
        cost_bbox = cost_bbox + jnp.abs(nb[k] - tgt[k])

    inter = None
    for k in range(3):
        amin = pcol[k] - pcol[3 + k] / 2.0
        amax = pcol[k] + pcol[3 + k] / 2.0
        bmin = tgt_bev[k] - tgt_bev[3 + k] / 2.0
        bmax = tgt_bev[k] + tgt_bev[3 + k] / 2.0
        d = jnp.maximum(jnp.minimum(amax, bmax) - jnp.maximum(amin, bmin),
                        0.0)
        inter = d if inter is None else inter * d
    va = (jnp.maximum(pcol[3], 1e-6) * jnp.maximum(pcol[4], 1e-6)
          * jnp.maximum(pcol[5], 1e-6))
    vb = (jnp.maximum(tgt_bev[3], 1e-6) * jnp.maximum(tgt_bev[4], 1e-6)
          * jnp.maximum(tgt_bev[5], 1e-6))
    iou = inter / (va + vb - inter + 1e-8)

    cost = (_COST_BBOX * cost_bbox + _COST_CLASS * cost_class
            - _COST_GIOU * iou
            + 100.0 * jnp.where(in_both, 0.0, 1.0))
    cost = cost + 10000.0 * (1.0 - fg)
    return cost, iou


def _extract_top10_counts(blk, sv, sc):
    vals, cnts = [], []
    for _ in range(_OTA_K):
        m = jnp.maximum(jnp.max(blk, axis=0, keepdims=True),
                        jnp.max(sv, axis=0, keepdims=True))
        eq_b = blk == m
        eq_s = sv == m
        c = (jnp.sum(jnp.where(eq_b, 1, 0), axis=0, keepdims=True)
             + jnp.sum(jnp.where(eq_s, sc, 0), axis=0, keepdims=True))
        vals.append(m)
        cnts.append(c)
        blk = jnp.where(eq_b, _NEG_INF, blk)
        sv = jnp.where(eq_s, _NEG_INF, sv)
    return vals, cnts


def _fused(logits_ref, pbox_ref, gtT_ref, labels_ref, gs_ref, isz_ref,
           off_ref, match_ref, cnt_ref, costs_ref, kq_ref, iv_ref, ii_ref,
           tvs_ref, tis_ref, cs_ref):
    ph = pl.program_id(0)
    pid = pl.program_id(1)
    nblk = pl.num_programs(1)
    R = logits_ref.shape[0]
    NQ, M = costs_ref.shape

    @pl.when(ph == 0)
    def _phase0():
        @pl.when(pid == 0)
        def _init():
            iv_ref[...] = jnp.full((16, M), _NEG_INF, jnp.float32)
            ii_ref[...] = jnp.zeros((16, M), jnp.int32)

        gs = (gs_ref[0], gs_ref[1], gs_ref[2])
        isz = (isz_ref[0], isz_ref[1], isz_ref[2])
        off = (off_ref[0], off_ref[1], off_ref[2])
        cost, iou = _cost_and_iou_block(logits_ref[...], pbox_ref[...],
                                        gtT_ref[...], labels_ref[0:1, :],
                                        gs, isz, off)
        costs_ref[pl.ds(pid * R, R), :] = cost

        ivals, icnts = _extract_top10_counts(iou, iv_ref[...], ii_ref[...])
        iv_ref[...] = jnp.concatenate(
            ivals + [jnp.full((16 - _OTA_K, M), _NEG_INF, jnp.float32)],
            axis=0)
        ii_ref[...] = jnp.concatenate(
            icnts + [jnp.zeros((16 - _OTA_K, M), jnp.int32)], axis=0)

        @pl.when(pid == nblk - 1)
        def _finish():
            s = jnp.zeros((1, M), jnp.float32)
            cum = jnp.zeros((1, M), jnp.int32)
            for t in range(_OTA_K):
                take = jnp.clip(_OTA_K - cum, 0, icnts[t])
                s = s + jnp.where(take > 0,
                                  ivals[t] * take.astype(jnp.float32), 0.0)
                cum = cum + take
            k = jnp.clip(s.astype(jnp.int32), 1, None)
            kq_ref[...] = jnp.broadcast_to(k - 1, (8, M))

    @pl.when(ph == 1)
    def _phase1():
        @pl.when(pid == 0)
        def _thresholds():
            cs_ref[...] = jnp.zeros((8, M), jnp.float32)
            km1 = kq_ref[0:1, :]
            kmax = jnp.max(km1) + 1
            iota = jax.lax.broadcasted_iota(jnp.int32, (NQ, M), 0)

            def body(t, carry):
                tvp, tip, tv, ti = carry
                c = costs_ref[...]
                after = (c > tvp) | ((c == tvp) & (iota > tip))
                m = jnp.min(jnp.where(after, c, _POS_INF), axis=0,
                            keepdims=True)
                mi = jnp.min(jnp.where(after & (c == m), iota, _BIG_I32),
                             axis=0, keepdims=True)
                sel = km1 == t
                return (m, mi, jnp.where(sel, m, tv),
                        jnp.where(sel, mi, ti))

            _, _, tv, ti = jax.lax.fori_loop(
                0, kmax, body,
                (jnp.full((1, M), _NEG_INF, jnp.float32),
                 jnp.full((1, M), -1, jnp.int32),
                 jnp.zeros((1, M), jnp.float32),
                 jnp.zeros((1, M), jnp.int32)))
            tvs_ref[...] = jnp.broadcast_to(tv, (8, M))
            tis_ref[...] = jnp.broadcast_to(ti, (8, M))

        cost = costs_ref[pl.ds(pid * R, R), :]
        ridx = jax.lax.broadcasted_iota(jnp.int32, (R, M), 0) + pid * R
        tv = tvs_ref[0:1, :]
        ti = tis_ref[0:1, :]

        m0 = jnp.where((cost < tv) | ((cost == tv) & (ridx <= ti)),
                       1.0, 0.0)
        ones_mm = jnp.ones((M, M), jnp.float32)
        am = jax.lax.dot_general(m0, ones_mm, (((1,), (0,)), ((), ())),
                                 preferred_element_type=jnp.float32)

        rowmin = jnp.min(cost, axis=1, keepdims=True)
        ismin = jnp.where(cost == rowmin, 1.0, 0.0)
        strict_lt = jnp.where(
            jax.lax.broadcasted_iota(jnp.int32, (M, M), 0)
            < jax.lax.broadcasted_iota(jnp.int32, (M, M), 1), 1.0, 0.0)
        prior = jax.lax.dot_general(ismin, strict_lt,
                                    (((1,), (0,)), ((), ())),
                                    preferred_element_type=jnp.float32)
        onehot = ismin * jnp.where(prior == 0.0, 1.0, 0.0)

        mat = jnp.where(am > 1.0, onehot, m0)
        match_ref[...] = mat
        cs_ref[0:1, :] = cs_ref[0:1, :] + jnp.sum(mat, axis=0,
                                                  keepdims=True)

        @pl.when(pid == nblk - 1)
        def _count():
            cnt = jnp.sum(
                jnp.where(cs_ref[0:1, :] > 0.0, jnp.ones((), jnp.int32),
                          jnp.zeros((), jnp.int32)), axis=1, keepdims=True)
            cnt_ref[...] = jnp.broadcast_to(cnt, (8, M))


def _pick_block(nq):
    for r in (2000, 1000, 800, 500, 400, 250, 200, 125, 100, 50, 40, 25,
              20, 10, 8, 5, 4, 2, 1):
        if nq % r == 0:
            return r
    return nq


def kernel(pred_logits, pred_boxes, gt_boxes, gt_labels, grid_size,
           image_size, offset):
    nq, _ = pred_logits.shape
    m = gt_boxes.shape[0]
    r = _pick_block(nq)
    nblk = nq // r

    gtT = jnp.concatenate(
        [gt_boxes.T.astype(jnp.float32), jnp.zeros((1, m), jnp.float32)],
        axis=0)
    labels2d = jnp.broadcast_to(
        gt_labels.astype(jnp.int32).reshape(1, m), (8, m))
    gs = grid_size.astype(jnp.float32)
    isz = image_size.astype(jnp.float32)
    off = offset.astype(jnp.float32)

    matching, cnt = pl.pallas_call(
        _fused,
        grid=(2, nblk),
        in_specs=[
            pl.BlockSpec((r, pred_logits.shape[1]), lambda p, b: (b, 0)),
            pl.BlockSpec((r, 7), lambda p, b: (b, 0)),
            pl.BlockSpec((8, m), lambda p, b: (0, 0)),
            pl.BlockSpec((8, m), lambda p, b: (0, 0)),
            pl.BlockSpec(memory_space=pltpu.SMEM),
            pl.BlockSpec(memory_space=pltpu.SMEM),
            pl.BlockSpec(memory_space=pltpu.SMEM),
        ],
        out_specs=[pl.BlockSpec((r, m), lambda p, b: (b * p, 0)),
                   pl.BlockSpec((8, m), lambda p, b: (0, 0))],
        out_shape=[jax.ShapeDtypeStruct((nq, m), jnp.float32),
                   jax.ShapeDtypeStruct((8, m), jnp.int32)],
        scratch_shapes=[pltpu.VMEM((nq, m), jnp.float32),
                        pltpu.VMEM((8, m), jnp.int32),
                        pltpu.VMEM((16, m), jnp.float32),
                        pltpu.VMEM((16, m), jnp.int32),
                        pltpu.VMEM((8, m), jnp.float32),
                        pltpu.VMEM((8, m), jnp.int32),
                        pltpu.VMEM((8, m), jnp.float32)],
    )(pred_logits, pred_boxes, gtT, labels2d, gs, isz, off)

    return matching, cnt[0, 0]


# speedup vs baseline: 17.6199x; 1.0111x over previous
import jax
import jax.numpy as jnp
from jax.experimental import pallas as pl
from jax.experimental.pallas import tpu as pltpu

_COST_CLASS = 1.0
_COST_BBOX = 1.0
_COST_GIOU = 1.0
_OTA_K = 10
_CENTER_RADIUS = 2.5
_ALPHA = 0.25

_POS_INF = float("inf")
_NEG_INF = float("-inf")
_BIG_I32 = 2**31 - 1


def _cost_and_iou_block(logits, pbox, gtT, labels_row, gs, isz, off):
    R, C = logits.shape
    M = labels_row.shape[1]

    p = jax.nn.sigmoid(logits)
    omp = 1.0 - p
    neg = (1.0 - _ALPHA) * (p * p) * (-jnp.log(1.0 - p + 1e-8))
    pos = _ALPHA * (omp * omp) * (-jnp.log(p + 1e-8))
    diff = pos - neg
    cpad = 16
    diff16 = jnp.concatenate(
        [diff, jnp.zeros((R, cpad - C), jnp.float32)], axis=1)
    onehot_lab = jnp.where(
        jax.lax.broadcasted_iota(jnp.int32, (cpad, M), 0) == labels_row,
        1.0, 0.0)
    cost_class = jax.lax.dot_general(
        diff16, onehot_lab, (((1,), (0,)), ((), ())),
        precision=jax.lax.Precision.HIGHEST,
        preferred_element_type=jnp.float32)

    pcol = [pbox[:, k:k + 1] for k in range(7)]
    grow = [gtT[k:k + 1, :] for k in range(7)]

    nb = [pcol[0] / gs[0], pcol[1] / gs[1], pcol[2] / gs[2],
          pcol[3] / gs[0], pcol[4] / gs[1], pcol[5] / gs[2], pcol[6]]
    tgt = [(grow[0] - off[0]) / isz[0], (grow[1] - off[1]) / isz[1],
           (grow[2] - off[2]) / isz[2],
           grow[3] / isz[0], grow[4] / isz[1], grow[5] / isz[2], grow[6]]
    tgt_bev = [tgt[0] * gs[0], tgt[1] * gs[1], tgt[2] * gs[2],
               tgt[3] * gs[0], tgt[4] * gs[1], tgt[5] * gs[2], tgt[6]]
    wx = nb[0] * isz[0]
    wy = nb[1] * isz[1]

    ca = jnp.cos(grow[6])
    sa = jnp.sin(grow[6])
    hx = grow[3] / 2.0
    hy = grow[4] / 2.0
    x0 = (ca * (-hx) + sa * (-hy)) + grow[0]
    y0 = (-sa * (-hx) + ca * (-hy)) + grow[1]
    x1 = (ca * (-hx) + sa * hy) + grow[0]
    y1 = (-sa * (-hx) + ca * hy) + grow[1]
    x3 = (ca * hx + sa * (-hy)) + grow[0]
    y3 = (-sa * hx + ca * (-hy)) + grow[1]

    ab0 = x1 - x0
    ab1 = y1 - y0
    ad0 = x3 - x0
    ad1 = y3 - y0
    ap0 = wx - x0
    ap1 = wy - y0
    abab = ab0 * ab0 + ab1 * ab1
    abap = ab0 * ap0 + ab1 * ap1
    adad = ad0 * ad0 + ad1 * ad1
    adap = ad0 * ap0 + ad1 * ap1
    in_boxes = ((abab >= abap) & (abap >= 0)
                & ((adad >= adap) & (adap >= 0)))

    dx = jnp.abs(wx - grow[0])
    dy = jnp.abs(wy - grow[1])
    in_centers = (dx < _CENTER_RADIUS) & (dy < _CENTER_RADIUS)
    fg = jnp.max(jnp.where(in_boxes | in_centers, 1.0, 0.0), axis=1,
                 keepdims=True)
    in_both = in_boxes & in_centers

    cost_bbox = jnp.abs(nb[0] - tgt[0])
    for k in range(1, 7):
        cost_bbox = cost_bbox + jnp.abs(nb[k] - tgt[k])

    inter = None
    for k in range(3):
        amin = pcol[k] - pcol[3 + k] / 2.0
        amax = pcol[k] + pcol[3 + k] / 2.0
        bmin = tgt_bev[k] - tgt_bev[3 + k] / 2.0
        bmax = tgt_bev[k] + tgt_bev[3 + k] / 2.0
        d = jnp.maximum(jnp.minimum(amax, bmax) - jnp.maximum(amin, bmin),
                        0.0)
        inter = d if inter is None else inter * d
    va = (jnp.maximum(pcol[3], 1e-6) * jnp.maximum(pcol[4], 1e-6)
          * jnp.maximum(pcol[5], 1e-6))
    vb = (jnp.maximum(tgt_bev[3], 1e-6) * jnp.maximum(tgt_bev[4], 1e-6)
          * jnp.maximum(tgt_bev[5], 1e-6))
    iou = inter / (va + vb - inter + 1e-8)

    cost = (_COST_BBOX * cost_bbox + _COST_CLASS * cost_class
            - _COST_GIOU * iou
            + 100.0 * jnp.where(in_both, 0.0, 1.0))
    cost = cost + 10000.0 * (1.0 - fg)
    return cost, iou


def _extract_top10_counts(blk, sv, sc):
    vals, cnts = [], []
    for _ in range(_OTA_K):
        m = jnp.maximum(jnp.max(blk, axis=0, keepdims=True),
                        jnp.max(sv, axis=0, keepdims=True))
        eq_b = blk == m
        eq_s = sv == m
        c = (jnp.sum(jnp.where(eq_b, 1, 0), axis=0, keepdims=True)
             + jnp.sum(jnp.where(eq_s, sc, 0), axis=0, keepdims=True))
        vals.append(m)
        cnts.append(c)
        blk = jnp.where(eq_b, _NEG_INF, blk)
        sv = jnp.where(eq_s, _NEG_INF, sv)
    return vals, cnts


def _fused(logits_ref, pbox_ref, gtT_ref, labels_ref, gs_ref, isz_ref,
           off_ref, match_ref, cnt_ref, costs_ref, kq_ref, iv_ref, ii_ref,
           tvs_ref, tis_ref, cs_ref):
    ph = pl.program_id(0)
    pid = pl.program_id(1)
    nblk = pl.num_programs(1)
    R = logits_ref.shape[0]
    NQ, M = costs_ref.shape

    @pl.when(ph == 0)
    def _phase0():
        @pl.when(pid == 0)
        def _init():
            iv_ref[...] = jnp.full((16, M), _NEG_INF, jnp.float32)
            ii_ref[...] = jnp.zeros((16, M), jnp.int32)

        gs = (gs_ref[0], gs_ref[1], gs_ref[2])
        isz = (isz_ref[0], isz_ref[1], isz_ref[2])
        off = (off_ref[0], off_ref[1], off_ref[2])
        cost, iou = _cost_and_iou_block(logits_ref[...], pbox_ref[...],
                                        gtT_ref[...], labels_ref[0:1, :],
                                        gs, isz, off)
        costs_ref[pl.ds(pid * R, R), :] = cost

        ivals, icnts = _extract_top10_counts(iou, iv_ref[...], ii_ref[...])
        iv_ref[...] = jnp.concatenate(
            ivals + [jnp.full((16 - _OTA_K, M), _NEG_INF, jnp.float32)],
            axis=0)
        ii_ref[...] = jnp.concatenate(
            icnts + [jnp.zeros((16 - _OTA_K, M), jnp.int32)], axis=0)

        @pl.when(pid == nblk - 1)
        def _finish():
            s = jnp.zeros((1, M), jnp.float32)
            cum = jnp.zeros((1, M), jnp.int32)
            for t in range(_OTA_K):
                take = jnp.clip(_OTA_K - cum, 0, icnts[t])
                s = s + jnp.where(take > 0,
                                  ivals[t] * take.astype(jnp.float32), 0.0)
                cum = cum + take
            k = jnp.clip(s.astype(jnp.int32), 1, None)
            kq_ref[...] = jnp.broadcast_to(k - 1, (8, M))

    @pl.when(ph == 1)
    def _phase1():
        @pl.when(pid == 0)
        def _thresholds():
            cs_ref[...] = jnp.zeros((8, M), jnp.float32)
            km1 = kq_ref[0:1, :]
            kmax = jnp.max(km1) + 1
            iota = jax.lax.broadcasted_iota(jnp.int32, (NQ, M), 0)

            def body(t, carry):
                tvp, tip, tv, ti = carry
                c = costs_ref[...]
                after = (c > tvp) | ((c == tvp) & (iota > tip))
                m = jnp.min(jnp.where(after, c, _POS_INF), axis=0,
                            keepdims=True)
                mi = jnp.min(jnp.where(after & (c == m), iota, _BIG_I32),
                             axis=0, keepdims=True)
                sel = km1 == t
                return (m, mi, jnp.where(sel, m, tv),
                        jnp.where(sel, mi, ti))

            _, _, tv, ti = jax.lax.fori_loop(
                0, kmax, body,
                (jnp.full((1, M), _NEG_INF, jnp.float32),
                 jnp.full((1, M), -1, jnp.int32),
                 jnp.zeros((1, M), jnp.float32),
                 jnp.zeros((1, M), jnp.int32)))
            tvs_ref[...] = jnp.broadcast_to(tv, (8, M))
            tis_ref[...] = jnp.broadcast_to(ti, (8, M))

        cost = costs_ref[pl.ds(pid * R, R), :]
        ridx = jax.lax.broadcasted_iota(jnp.int32, (R, M), 0) + pid * R
        tv = tvs_ref[0:1, :]
        ti = tis_ref[0:1, :]

        m0 = jnp.where((cost < tv) | ((cost == tv) & (ridx <= ti)),
                       1.0, 0.0)
        ones_mm = jnp.ones((M, M), jnp.float32)
        am = jax.lax.dot_general(m0, ones_mm, (((1,), (0,)), ((), ())),
                                 preferred_element_type=jnp.float32)

        rowmin = jnp.min(cost, axis=1, keepdims=True)
        ismin = jnp.where(cost == rowmin, 1.0, 0.0)
        strict_lt = jnp.where(
            jax.lax.broadcasted_iota(jnp.int32, (M, M), 0)
            < jax.lax.broadcasted_iota(jnp.int32, (M, M), 1), 1.0, 0.0)
        prior = jax.lax.dot_general(ismin, strict_lt,
                                    (((1,), (0,)), ((), ())),
                                    preferred_element_type=jnp.float32)
        onehot = ismin * jnp.where(prior == 0.0, 1.0, 0.0)

        mat = jnp.where(am > 1.0, onehot, m0)
        match_ref[...] = mat
        cs_ref[0:1, :] = cs_ref[0:1, :] + jnp.sum(mat, axis=0,
                                                  keepdims=True)

        @pl.when(pid == nblk - 1)
        def _count():
            cnt = jnp.sum(
                jnp.where(cs_ref[0:1, :] > 0.0, jnp.ones((), jnp.int32),
                          jnp.zeros((), jnp.int32)), axis=1, keepdims=True)
            cnt_ref[...] = jnp.broadcast_to(cnt, (8, M))


def _pick_block(nq):
    for r in (4000, 2000, 1000, 800, 500, 400, 250, 200, 125, 100, 50, 40,
              25, 20, 10, 8, 5, 4, 2, 1):
        if nq % r == 0:
            return r
    return nq


def kernel(pred_logits, pred_boxes, gt_boxes, gt_labels, grid_size,
           image_size, offset):
    nq, _ = pred_logits.shape
    m = gt_boxes.shape[0]
    r = _pick_block(nq)
    nblk = nq // r

    gtT = jnp.concatenate(
        [gt_boxes.T.astype(jnp.float32), jnp.zeros((1, m), jnp.float32)],
        axis=0)
    labels2d = jnp.broadcast_to(
        gt_labels.astype(jnp.int32).reshape(1, m), (8, m))
    gs = grid_size.astype(jnp.float32)
    isz = image_size.astype(jnp.float32)
    off = offset.astype(jnp.float32)

    matching, cnt = pl.pallas_call(
        _fused,
        grid=(2, nblk),
        in_specs=[
            pl.BlockSpec((r, pred_logits.shape[1]), lambda p, b: (b, 0)),
            pl.BlockSpec((r, 7), lambda p, b: (b, 0)),
            pl.BlockSpec((8, m), lambda p, b: (0, 0)),
            pl.BlockSpec((8, m), lambda p, b: (0, 0)),
            pl.BlockSpec(memory_space=pltpu.SMEM),
            pl.BlockSpec(memory_space=pltpu.SMEM),
            pl.BlockSpec(memory_space=pltpu.SMEM),
        ],
        out_specs=[pl.BlockSpec((r, m), lambda p, b: (b * p, 0)),
                   pl.BlockSpec((8, m), lambda p, b: (0, 0))],
        out_shape=[jax.ShapeDtypeStruct((nq, m), jnp.float32),
                   jax.ShapeDtypeStruct((8, m), jnp.int32)],
        scratch_shapes=[pltpu.VMEM((nq, m), jnp.float32),
                        pltpu.VMEM((8, m), jnp.int32),
                        pltpu.VMEM((16, m), jnp.float32),
                        pltpu.VMEM((16, m), jnp.int32),
                        pltpu.VMEM((8, m), jnp.float32),
                        pltpu.VMEM((8, m), jnp.int32),
                        pltpu.VMEM((8, m), jnp.float32)],
    )(pred_logits, pred_boxes, gtT, labels2d, gs, isz, off)

    return matching, cnt[0, 0]


# speedup vs baseline: 17.9232x; 1.0172x over previous
import jax
import jax.numpy as jnp
from jax.experimental import pallas as pl
from jax.experimental.pallas import tpu as pltpu

_COST_CLASS = 1.0
_COST_BBOX = 1.0
_COST_GIOU = 1.0
_OTA_K = 10
_CENTER_RADIUS = 2.5
_ALPHA = 0.25

_POS_INF = float("inf")
_NEG_INF = float("-inf")
_BIG_I32 = 2**31 - 1


def _cost_and_iou_block(logits, pbox, gtT, labels_row, gs, isz, off):
    R, C = logits.shape
    M = labels_row.shape[1]

    p = jax.nn.sigmoid(logits)
    omp = 1.0 - p
    neg = (1.0 - _ALPHA) * (p * p) * (-jnp.log(1.0 - p + 1e-8))
    pos = _ALPHA * (omp * omp) * (-jnp.log(p + 1e-8))
    diff = pos - neg
    cpad = 16
    diff16 = jnp.concatenate(
        [diff, jnp.zeros((R, cpad - C), jnp.float32)], axis=1)
    onehot_lab = jnp.where(
        jax.lax.broadcasted_iota(jnp.int32, (cpad, M), 0) == labels_row,
        1.0, 0.0)
    cost_class = jax.lax.dot_general(
        diff16, onehot_lab, (((1,), (0,)), ((), ())),
        precision=jax.lax.Precision.HIGHEST,
        preferred_element_type=jnp.float32)

    pcol = [pbox[:, k:k + 1] for k in range(7)]
    grow = [gtT[k:k + 1, :] for k in range(7)]

    nb = [pcol[0] / gs[0], pcol[1] / gs[1], pcol[2] / gs[2],
          pcol[3] / gs[0], pcol[4] / gs[1], pcol[5] / gs[2], pcol[6]]
    tgt = [(grow[0] - off[0]) / isz[0], (grow[1] - off[1]) / isz[1],
           (grow[2] - off[2]) / isz[2],
           grow[3] / isz[0], grow[4] / isz[1], grow[5] / isz[2], grow[6]]
    tgt_bev = [tgt[0] * gs[0], tgt[1] * gs[1], tgt[2] * gs[2],
               tgt[3] * gs[0], tgt[4] * gs[1], tgt[5] * gs[2], tgt[6]]
    wx = nb[0] * isz[0]
    wy = nb[1] * isz[1]

    ca = jnp.cos(grow[6])
    sa = jnp.sin(grow[6])
    hx = grow[3] / 2.0
    hy = grow[4] / 2.0
    x0 = (ca * (-hx) + sa * (-hy)) + grow[0]
    y0 = (-sa * (-hx) + ca * (-hy)) + grow[1]
    x1 = (ca * (-hx) + sa * hy) + grow[0]
    y1 = (-sa * (-hx) + ca * hy) + grow[1]
    x3 = (ca * hx + sa * (-hy)) + grow[0]
    y3 = (-sa * hx + ca * (-hy)) + grow[1]

    ab0 = x1 - x0
    ab1 = y1 - y0
    ad0 = x3 - x0
    ad1 = y3 - y0
    ap0 = wx - x0
    ap1 = wy - y0
    abab = ab0 * ab0 + ab1 * ab1
    abap = ab0 * ap0 + ab1 * ap1
    adad = ad0 * ad0 + ad1 * ad1
    adap = ad0 * ap0 + ad1 * ap1
    in_boxes = ((abab >= abap) & (abap >= 0)
                & ((adad >= adap) & (adap >= 0)))

    dx = jnp.abs(wx - grow[0])
    dy = jnp.abs(wy - grow[1])
    in_centers = (dx < _CENTER_RADIUS) & (dy < _CENTER_RADIUS)
    inany = jnp.where(in_boxes | in_centers, 1.0, 0.0)
    fgcnt = jax.lax.dot_general(
        inany, jnp.ones((M, M), jnp.float32), (((1,), (0,)), ((), ())),
        preferred_element_type=jnp.float32)
    in_both = in_boxes & in_centers

    cost_bbox = jnp.abs(nb[0] - tgt[0])
    for k in range(1, 7):
        cost_bbox = cost_bbox + jnp.abs(nb[k] - tgt[k])

    inter = None
    for k in range(3):
        amin = pcol[k] - pcol[3 + k] / 2.0
        amax = pcol[k] + pcol[3 + k] / 2.0
        bmin = tgt_bev[k] - tgt_bev[3 + k] / 2.0
        bmax = tgt_bev[k] + tgt_bev[3 + k] / 2.0
        d = jnp.maximum(jnp.minimum(amax, bmax) - jnp.maximum(amin, bmin),
                        0.0)
        inter = d if inter is None else inter * d
    va = (jnp.maximum(pcol[3], 1e-6) * jnp.maximum(pcol[4], 1e-6)
          * jnp.maximum(pcol[5], 1e-6))
    vb = (jnp.maximum(tgt_bev[3], 1e-6) * jnp.maximum(tgt_bev[4], 1e-6)
          * jnp.maximum(tgt_bev[5], 1e-6))
    iou = inter / (va + vb - inter + 1e-8)

    cost = (_COST_BBOX * cost_bbox + _COST_CLASS * cost_class
            - _COST_GIOU * iou
            + 100.0 * jnp.where(in_both, 0.0, 1.0))
    cost = cost + 10000.0 * jnp.where(fgcnt > 0.0, 0.0, 1.0)
    return cost, iou


def _extract_top10_counts(blk, sv, sc):
    vals, cnts = [], []
    for _ in range(_OTA_K):
        m = jnp.maximum(jnp.max(blk, axis=0, keepdims=True),
                        jnp.max(sv, axis=0, keepdims=True))
        eq_b = blk == m
        eq_s = sv == m
        c = (jnp.sum(jnp.where(eq_b, 1, 0), axis=0, keepdims=True)
             + jnp.sum(jnp.where(eq_s, sc, 0), axis=0, keepdims=True))
        vals.append(m)
        cnts.append(c)
        blk = jnp.where(eq_b, _NEG_INF, blk)
        sv = jnp.where(eq_s, _NEG_INF, sv)
    return vals, cnts


def _fused(logits_ref, pbox_ref, gtT_ref, labels_ref, gs_ref, isz_ref,
           off_ref, match_ref, cnt_ref, costs_ref, kq_ref, iv_ref, ii_ref,
           tvs_ref, tis_ref, cs_ref):
    ph = pl.program_id(0)
    pid = pl.program_id(1)
    nblk = pl.num_programs(1)
    R = logits_ref.shape[0]
    NQ, M = costs_ref.shape

    @pl.when(ph == 0)
    def _phase0():
        @pl.when(pid == 0)
        def _init():
            iv_ref[...] = jnp.full((16, M), _NEG_INF, jnp.float32)
            ii_ref[...] = jnp.zeros((16, M), jnp.int32)

        gs = (gs_ref[0], gs_ref[1], gs_ref[2])
        isz = (isz_ref[0], isz_ref[1], isz_ref[2])
        off = (off_ref[0], off_ref[1], off_ref[2])
        cost, iou = _cost_and_iou_block(logits_ref[...], pbox_ref[...],
                                        gtT_ref[...], labels_ref[0:1, :],
                                        gs, isz, off)
        costs_ref[pl.ds(pid * R, R), :] = cost

        ivals, icnts = _extract_top10_counts(iou, iv_ref[...], ii_ref[...])
        iv_ref[...] = jnp.concatenate(
            ivals + [jnp.full((16 - _OTA_K, M), _NEG_INF, jnp.float32)],
            axis=0)
        ii_ref[...] = jnp.concatenate(
            icnts + [jnp.zeros((16 - _OTA_K, M), jnp.int32)], axis=0)

        @pl.when(pid == nblk - 1)
        def _finish():
            s = jnp.zeros((1, M), jnp.float32)
            cum = jnp.zeros((1, M), jnp.int32)
            for t in range(_OTA_K):
                take = jnp.clip(_OTA_K - cum, 0, icnts[t])
                s = s + jnp.where(take > 0,
                                  ivals[t] * take.astype(jnp.float32), 0.0)
                cum = cum + take
            k = jnp.clip(s.astype(jnp.int32), 1, None)
            kq_ref[...] = jnp.broadcast_to(k - 1, (8, M))

    @pl.when(ph == 1)
    def _phase1():
        @pl.when(pid == 0)
        def _thresholds():
            cs_ref[...] = jnp.zeros((8, M), jnp.float32)
            km1 = kq_ref[0:1, :]
            kmax = jnp.max(km1) + 1
            iota = jax.lax.broadcasted_iota(jnp.int32, (NQ, M), 0)

            def body(t, carry):
                tvp, tip, tv, ti = carry
                c = costs_ref[...]
                after = (c > tvp) | ((c == tvp) & (iota > tip))
                m = jnp.min(jnp.where(after, c, _POS_INF), axis=0,
                            keepdims=True)
                mi = jnp.min(jnp.where(after & (c == m), iota, _BIG_I32),
                             axis=0, keepdims=True)
                sel = km1 == t
                return (m, mi, jnp.where(sel, m, tv),
                        jnp.where(sel, mi, ti))

            _, _, tv, ti = jax.lax.fori_loop(
                0, kmax, body,
                (jnp.full((1, M), _NEG_INF, jnp.float32),
                 jnp.full((1, M), -1, jnp.int32),
                 jnp.zeros((1, M), jnp.float32),
                 jnp.zeros((1, M), jnp.int32)))
            tvs_ref[...] = jnp.broadcast_to(tv, (8, M))
            tis_ref[...] = jnp.broadcast_to(ti, (8, M))

        cost = costs_ref[pl.ds(pid * R, R), :]
        ridx = jax.lax.broadcasted_iota(jnp.int32, (R, M), 0) + pid * R
        tv = tvs_ref[0:1, :]
        ti = tis_ref[0:1, :]

        m0 = jnp.where((cost < tv) | ((cost == tv) & (ridx <= ti)),
                       1.0, 0.0)
        ones_mm = jnp.ones((M, M), jnp.float32)
        am = jax.lax.dot_general(m0, ones_mm, (((1,), (0,)), ((), ())),
                                 preferred_element_type=jnp.float32)

        rowmin = jnp.min(cost, axis=1, keepdims=True)
        ismin = jnp.where(cost == rowmin, 1.0, 0.0)
        strict_lt = jnp.where(
            jax.lax.broadcasted_iota(jnp.int32, (M, M), 0)
            < jax.lax.broadcasted_iota(jnp.int32, (M, M), 1), 1.0, 0.0)
        prior = jax.lax.dot_general(ismin, strict_lt,
                                    (((1,), (0,)), ((), ())),
                                    preferred_element_type=jnp.float32)
        onehot = ismin * jnp.where(prior == 0.0, 1.0, 0.0)

        mat = jnp.where(am > 1.0, onehot, m0)
        match_ref[...] = mat
        cs_ref[0:1, :] = cs_ref[0:1, :] + jnp.sum(mat, axis=0,
                                                  keepdims=True)

        @pl.when(pid == nblk - 1)
        def _count():
            cnt = jnp.sum(
                jnp.where(cs_ref[0:1, :] > 0.0, jnp.ones((), jnp.int32),
                          jnp.zeros((), jnp.int32)), axis=1, keepdims=True)
            cnt_ref[...] = jnp.broadcast_to(cnt, (8, M))


def _pick_block(nq):
    for r in (4000, 2000, 1000, 800, 500, 400, 250, 200, 125, 100, 50, 40,
              25, 20, 10, 8, 5, 4, 2, 1):
        if nq % r == 0:
            return r
    return nq


def kernel(pred_logits, pred_boxes, gt_boxes, gt_labels, grid_size,
           image_size, offset):
    nq, _ = pred_logits.shape
    m = gt_boxes.shape[0]
    r = _pick_block(nq)
    nblk = nq // r

    gtT = jnp.concatenate(
        [gt_boxes.T.astype(jnp.float32), jnp.zeros((1, m), jnp.float32)],
        axis=0)
    labels2d = jnp.broadcast_to(
        gt_labels.astype(jnp.int32).reshape(1, m), (8, m))
    gs = grid_size.astype(jnp.float32)
    isz = image_size.astype(jnp.float32)
    off = offset.astype(jnp.float32)

    matching, cnt = pl.pallas_call(
        _fused,
        grid=(2, nblk),
        in_specs=[
            pl.BlockSpec((r, pred_logits.shape[1]), lambda p, b: (b, 0)),
            pl.BlockSpec((r, 7), lambda p, b: (b, 0)),
            pl.BlockSpec((8, m), lambda p, b: (0, 0)),
            pl.BlockSpec((8, m), lambda p, b: (0, 0)),
            pl.BlockSpec(memory_space=pltpu.SMEM),
            pl.BlockSpec(memory_space=pltpu.SMEM),
            pl.BlockSpec(memory_space=pltpu.SMEM),
        ],
        out_specs=[pl.BlockSpec((r, m), lambda p, b: (b * p, 0)),
                   pl.BlockSpec((8, m), lambda p, b: (0, 0))],
        out_shape=[jax.ShapeDtypeStruct((nq, m), jnp.float32),
                   jax.ShapeDtypeStruct((8, m), jnp.int32)],
        scratch_shapes=[pltpu.VMEM((nq, m), jnp.float32),
                        pltpu.VMEM((8, m), jnp.int32),
                        pltpu.VMEM((16, m), jnp.float32),
                        pltpu.VMEM((16, m), jnp.int32),
                        pltpu.VMEM((8, m), jnp.float32),
                        pltpu.VMEM((8, m), jnp.int32),
                        pltpu.VMEM((8, m), jnp.float32)],
    )(pred_logits, pred_boxes, gtT, labels2d, gs, isz, off)

    return matching, cnt[0, 0]
